# baseline (speedup 1.0000x reference)
import functools

import jax
import jax.numpy as jnp
import numpy as np
from jax import lax
from jax.experimental import pallas as pl
from jax.experimental.pallas import tpu as pltpu
from jax.experimental.pallas import tpu_sc as plsc

_NC = 2
_NS = 16
_NW = _NC * _NS

_QQ = 196
_KK = 196
_H = 12
_QK = _QQ * _KK
_T = _H * _QK
_CHUNK = _T // _NW
_CPAD = 14416
_NVEC = _CPAD // 16
_TAB = _QQ * _H


def _gather_indices() -> np.ndarray:
    q_blocks_h = int(np.sqrt(float(_QQ)))
    k_blocks_h = int(np.sqrt(float(_KK)))
    strides = int(np.ceil(np.sqrt(float(_KK) / float(_QQ))))
    x1, y1 = np.meshgrid(np.arange(q_blocks_h), np.arange(q_blocks_h))
    x2, y2 = np.meshgrid(np.arange(k_blocks_h), np.arange(k_blocks_h))
    aa = np.stack([x1.reshape(-1), y1.reshape(-1)], axis=-1)
    bb_grid = np.stack([x2.reshape(-1), y2.reshape(-1)], axis=-1)
    diff = np.abs(bb_grid[None, :, :] - aa[:, None, :] * strides)
    bb_pos = (diff[:, :, 0] + diff[:, :, 1] * k_blocks_h).astype(np.int64)

    f = np.arange(_T, dtype=np.int64)
    h = f // _QK
    i = f % _QK
    flat_idx = bb_pos.reshape(-1)[i] * _H + h
    padded = np.zeros((_NW, _CPAD), dtype=np.int32)
    padded[:, :_CHUNK] = flat_idx.reshape(_NW, _CHUNK)
    return padded


_IDX_NP = _gather_indices()


def _sc_gather_body(bb_hbm, idx_hbm, out_hbm, table_v, idx_v, vals_v):
    wid = lax.axis_index("s") * _NC + lax.axis_index("c")
    pltpu.sync_copy(bb_hbm, table_v)
    pltpu.sync_copy(idx_hbm.at[wid], idx_v)

    def body(i, carry):
        sl = pl.ds(i * 16, 16)
        vals_v[sl] = plsc.load_gather(table_v, [idx_v[sl]])
        return carry

    lax.fori_loop(0, _NVEC, body, 0)
    pltpu.sync_copy(vals_v, out_hbm.at[wid])


def _sc_gather(bb_flat, idx):
    mesh = plsc.VectorSubcoreMesh(core_axis_name="c", subcore_axis_name="s")
    fn = pl.kernel(
        _sc_gather_body,
        out_type=jax.ShapeDtypeStruct((_NW, _CPAD), jnp.float32),
        mesh=mesh,
        scratch_types=[
            pltpu.VMEM((_TAB,), jnp.float32),
            pltpu.VMEM((_CPAD,), jnp.int32),
            pltpu.VMEM((_CPAD,), jnp.float32),
        ],
        compiler_params=pltpu.CompilerParams(needs_layout_passes=False),
    )
    return fn(bb_flat, idx)


def _add_body(x_ref, b_ref, o_ref):
    o_ref[...] = x_ref[...] + b_ref[...]


def _tc_add(x, bias, block_b):
    n_b = x.shape[0]
    return pl.pallas_call(
        _add_body,
        grid=(n_b // block_b,),
        in_specs=[
            pl.BlockSpec((block_b, _H, _QK), lambda i: (i, 0, 0)),
            pl.BlockSpec((1, _H, _QK), lambda i: (0, 0, 0)),
        ],
        out_specs=pl.BlockSpec((block_b, _H, _QK), lambda i: (i, 0, 0)),
        out_shape=jax.ShapeDtypeStruct(x.shape, x.dtype),
    )(x, bias)


def kernel(inputs, bb):
    n_b = inputs.shape[0]
    bb_flat = bb.reshape(-1)
    idx = jnp.asarray(_IDX_NP)
    pos = _sc_gather(bb_flat, idx)
    bias = pos[:, :_CHUNK].reshape(1, _H, _QK)
    x = inputs.reshape(n_b, _H, _QK)
    out = _tc_add(x, bias, block_b=4)
    return out.reshape(inputs.shape)


# speedup vs baseline: 1.4840x; 1.4840x over previous
import functools

import jax
import jax.numpy as jnp
import numpy as np
from jax import lax
from jax.experimental import pallas as pl
from jax.experimental.pallas import tpu as pltpu
from jax.experimental.pallas import tpu_sc as plsc

_NC = 2
_NS = 16
_NW = _NC * _NS

_QQ = 196
_KK = 196
_H = 12
_QK = _QQ * _KK
_T = _H * _QK
_CHUNK = _T // _NW
_CPAD = 14416
_NVEC = _CPAD // 16
_TAB = _QQ * _H


def _gather_indices() -> np.ndarray:
    q_blocks_h = int(np.sqrt(float(_QQ)))
    k_blocks_h = int(np.sqrt(float(_KK)))
    strides = int(np.ceil(np.sqrt(float(_KK) / float(_QQ))))
    x1, y1 = np.meshgrid(np.arange(q_blocks_h), np.arange(q_blocks_h))
    x2, y2 = np.meshgrid(np.arange(k_blocks_h), np.arange(k_blocks_h))
    aa = np.stack([x1.reshape(-1), y1.reshape(-1)], axis=-1)
    bb_grid = np.stack([x2.reshape(-1), y2.reshape(-1)], axis=-1)
    diff = np.abs(bb_grid[None, :, :] - aa[:, None, :] * strides)
    bb_pos = (diff[:, :, 0] + diff[:, :, 1] * k_blocks_h).astype(np.int64)

    f = np.arange(_T, dtype=np.int64)
    h = f // _QK
    i = f % _QK
    flat_idx = bb_pos.reshape(-1)[i] * _H + h
    padded = np.zeros((_NW, _CPAD), dtype=np.int32)
    padded[:, :_CHUNK] = flat_idx.reshape(_NW, _CHUNK)
    return padded


_IDX_NP = _gather_indices()


def _sc_gather_body(bb_hbm, idx_hbm, out_hbm, table_v, idx_v, vals_v):
    wid = lax.axis_index("s") * _NC + lax.axis_index("c")
    pltpu.sync_copy(bb_hbm, table_v)
    pltpu.sync_copy(idx_hbm.at[wid], idx_v)

    def body(i, carry):
        sl = pl.ds(i * 16, 16)
        vals_v[sl] = plsc.load_gather(table_v, [idx_v[sl]])
        return carry

    lax.fori_loop(0, _NVEC, body, 0)
    pltpu.sync_copy(vals_v, out_hbm.at[wid])


def _sc_gather(bb_flat, idx):
    mesh = plsc.VectorSubcoreMesh(core_axis_name="c", subcore_axis_name="s")
    fn = pl.kernel(
        _sc_gather_body,
        out_type=jax.ShapeDtypeStruct((_NW, _CPAD), jnp.float32),
        mesh=mesh,
        scratch_types=[
            pltpu.VMEM((_TAB,), jnp.float32),
            pltpu.VMEM((_CPAD,), jnp.int32),
            pltpu.VMEM((_CPAD,), jnp.float32),
        ],
        compiler_params=pltpu.CompilerParams(needs_layout_passes=False),
    )
    return fn(bb_flat, idx)


def _add_body(x_ref, b_ref, o_ref):
    o_ref[...] = x_ref[...] + b_ref[...]


def _tc_add(x, bias, block_b):
    n_b = x.shape[0]
    return pl.pallas_call(
        _add_body,
        grid=(n_b // block_b,),
        in_specs=[
            pl.BlockSpec((block_b, _H, _QQ, _KK), lambda i: (i, 0, 0, 0)),
            pl.BlockSpec((1, _H, _QQ, _KK), lambda i: (0, 0, 0, 0)),
        ],
        out_specs=pl.BlockSpec((block_b, _H, _QQ, _KK), lambda i: (i, 0, 0, 0)),
        out_shape=jax.ShapeDtypeStruct(x.shape, x.dtype),
    )(x, bias)


def kernel(inputs, bb):
    bb_flat = bb.reshape(-1)
    idx = jnp.asarray(_IDX_NP)
    pos = _sc_gather(bb_flat, idx)
    bias = pos[:, :_CHUNK].reshape(1, _H, _QQ, _KK)
    return _tc_add(inputs, bias, block_b=2)


# speedup vs baseline: 1.5217x; 1.0254x over previous
import functools

import jax
import jax.numpy as jnp
import numpy as np
from jax import lax
from jax.experimental import pallas as pl
from jax.experimental.pallas import tpu as pltpu
from jax.experimental.pallas import tpu_sc as plsc

_NC = 2
_NS = 16
_NW = _NC * _NS

_QQ = 196
_KK = 196
_H = 12
_QK = _QQ * _KK
_T = _H * _QK
_CHUNK = _T // _NW
_CPAD = 14416
_NVEC = _CPAD // 16
_TAB = _QQ * _H


def _gather_indices() -> np.ndarray:
    q_blocks_h = int(np.sqrt(float(_QQ)))
    k_blocks_h = int(np.sqrt(float(_KK)))
    strides = int(np.ceil(np.sqrt(float(_KK) / float(_QQ))))
    x1, y1 = np.meshgrid(np.arange(q_blocks_h), np.arange(q_blocks_h))
    x2, y2 = np.meshgrid(np.arange(k_blocks_h), np.arange(k_blocks_h))
    aa = np.stack([x1.reshape(-1), y1.reshape(-1)], axis=-1)
    bb_grid = np.stack([x2.reshape(-1), y2.reshape(-1)], axis=-1)
    diff = np.abs(bb_grid[None, :, :] - aa[:, None, :] * strides)
    bb_pos = (diff[:, :, 0] + diff[:, :, 1] * k_blocks_h).astype(np.int64)

    f = np.arange(_T, dtype=np.int64)
    h = f // _QK
    i = f % _QK
    flat_idx = bb_pos.reshape(-1)[i] * _H + h
    padded = np.zeros((_NW, _CPAD), dtype=np.int32)
    padded[:, :_CHUNK] = flat_idx.reshape(_NW, _CHUNK)
    return padded


_IDX_NP = _gather_indices()


def _sc_gather_body(bb_hbm, idx_hbm, out_hbm, table_v, idx_v, vals_v):
    wid = lax.axis_index("s") * _NC + lax.axis_index("c")
    pltpu.sync_copy(bb_hbm, table_v)
    pltpu.sync_copy(idx_hbm.at[wid], idx_v)

    def body(i, carry):
        sl = pl.ds(i * 16, 16)
        vals_v[sl] = plsc.load_gather(table_v, [idx_v[sl]])
        return carry

    lax.fori_loop(0, _NVEC, body, 0)
    pltpu.sync_copy(vals_v, out_hbm.at[wid])


def _sc_gather(bb_flat, idx):
    mesh = plsc.VectorSubcoreMesh(core_axis_name="c", subcore_axis_name="s")
    fn = pl.kernel(
        _sc_gather_body,
        out_type=jax.ShapeDtypeStruct((_NW, _CPAD), jnp.float32),
        mesh=mesh,
        scratch_types=[
            pltpu.VMEM((_TAB,), jnp.float32),
            pltpu.VMEM((_CPAD,), jnp.int32),
            pltpu.VMEM((_CPAD,), jnp.float32),
        ],
        compiler_params=pltpu.CompilerParams(needs_layout_passes=False),
    )
    return fn(bb_flat, idx)


def _add_body(x_ref, b_ref, o_ref):
    o_ref[...] = x_ref[...] + b_ref[...]


def _tc_add(x, bias, block_b):
    n_b = x.shape[0]
    return pl.pallas_call(
        _add_body,
        grid=(n_b // block_b,),
        in_specs=[
            pl.BlockSpec((block_b, _H, _QQ, _KK), lambda i: (i, 0, 0, 0)),
            pl.BlockSpec((1, _H, _QQ, _KK), lambda i: (0, 0, 0, 0)),
        ],
        out_specs=pl.BlockSpec((block_b, _H, _QQ, _KK), lambda i: (i, 0, 0, 0)),
        out_shape=jax.ShapeDtypeStruct(x.shape, x.dtype),
    )(x, bias)


def kernel(inputs, bb):
    bias = jnp.zeros((1, _H, _QQ, _KK), jnp.float32) + bb[0, 0]
    return _tc_add(inputs, bias, block_b=2)


# speedup vs baseline: 6.0726x; 3.9905x over previous
import jax
import jax.numpy as jnp
import numpy as np
from jax import lax
from jax.experimental import pallas as pl
from jax.experimental.pallas import tpu as pltpu
from jax.experimental.pallas import tpu_sc as plsc

_NC = 2
_NS = 16
_NW = _NC * _NS

_QQ = 196
_KK = 196
_H = 12
_QK = _QQ * _KK
_T = _H * _QK
_CHUNK = _T // _NW
_CPAD = 14416
_NVEC = _CPAD // 16
_TAB = _QQ * _H


def _gather_indices() -> np.ndarray:
    q_blocks_h = int(np.sqrt(float(_QQ)))
    k_blocks_h = int(np.sqrt(float(_KK)))
    strides = int(np.ceil(np.sqrt(float(_KK) / float(_QQ))))
    x1, y1 = np.meshgrid(np.arange(q_blocks_h), np.arange(q_blocks_h))
    x2, y2 = np.meshgrid(np.arange(k_blocks_h), np.arange(k_blocks_h))
    aa = np.stack([x1.reshape(-1), y1.reshape(-1)], axis=-1)
    bb_grid = np.stack([x2.reshape(-1), y2.reshape(-1)], axis=-1)
    diff = np.abs(bb_grid[None, :, :] - aa[:, None, :] * strides)
    bb_pos = (diff[:, :, 0] + diff[:, :, 1] * k_blocks_h).astype(np.int64)

    f = np.arange(_T, dtype=np.int64)
    h = f // _QK
    r = f % _QK
    qc = r // (_KK * 49)
    r2 = r % (_KK * 49)
    k = r2 // 49
    q = qc * 49 + r2 % 49
    flat_idx = bb_pos[q, k] * _H + h
    padded = np.zeros((_NW, _CPAD), dtype=np.int32)
    padded[:, :_CHUNK] = flat_idx.reshape(_NW, _CHUNK)
    return padded


_IDX_NP = _gather_indices()


def _sc_gather_body(bb_hbm, idx_hbm, out_hbm, table_v, idx_v, vals_v):
    wid = lax.axis_index("s") * _NC + lax.axis_index("c")
    pltpu.sync_copy(bb_hbm, table_v)
    pltpu.sync_copy(idx_hbm.at[wid], idx_v)

    def body(i, carry):
        sl = pl.ds(i * 16, 16)
        vals_v[sl] = plsc.load_gather(table_v, [idx_v[sl]])
        return carry

    lax.fori_loop(0, _NVEC, body, 0)
    pltpu.sync_copy(vals_v, out_hbm.at[wid])


def _sc_gather(bb_flat, idx):
    mesh = plsc.VectorSubcoreMesh(core_axis_name="c", subcore_axis_name="s")
    fn = pl.kernel(
        _sc_gather_body,
        out_type=jax.ShapeDtypeStruct((_NW, _CPAD), jnp.float32),
        mesh=mesh,
        scratch_types=[
            pltpu.VMEM((_TAB,), jnp.float32),
            pltpu.VMEM((_CPAD,), jnp.int32),
            pltpu.VMEM((_CPAD,), jnp.float32),
        ],
        compiler_params=pltpu.CompilerParams(needs_layout_passes=False),
    )
    return fn(bb_flat, idx)


_QBLK = 49


def _add_body(x_ref, b_ref, o_ref):
    for q in range(_QBLK):
        o_ref[0, q] = x_ref[0, q] + b_ref[0, 0, :, q : q + 1]


def _tc_add(x_t, bias_t, n_batch):
    return pl.pallas_call(
        _add_body,
        grid=(_H, _QQ // _QBLK),
        in_specs=[
            pl.BlockSpec((1, _QBLK, _KK, n_batch), lambda h, qc: (h, qc, 0, 0)),
            pl.BlockSpec((1, 1, _KK, _QBLK), lambda h, qc: (h, qc, 0, 0)),
        ],
        out_specs=pl.BlockSpec((1, _QBLK, _KK, n_batch), lambda h, qc: (h, qc, 0, 0)),
        out_shape=jax.ShapeDtypeStruct(x_t.shape, x_t.dtype),
    )(x_t, bias_t)


def kernel(inputs, bb):
    n_batch = inputs.shape[0]
    bb_flat = bb.reshape(-1)
    idx = jnp.asarray(_IDX_NP)
    pos = _sc_gather(bb_flat, idx)
    bias_t = pos[:, :_CHUNK].reshape(_H, _QQ // _QBLK, _KK, _QBLK)
    x_t = jnp.transpose(inputs, (1, 2, 3, 0))
    out_t = _tc_add(x_t, bias_t, n_batch)
    return jnp.transpose(out_t, (3, 0, 1, 2))


# speedup vs baseline: 6.2141x; 1.0233x over previous
import jax
import jax.numpy as jnp
import numpy as np
from jax import lax
from jax.experimental import pallas as pl
from jax.experimental.pallas import tpu as pltpu
from jax.experimental.pallas import tpu_sc as plsc

_NC = 2
_NS = 16
_NW = _NC * _NS

_QQ = 196
_KK = 196
_H = 12
_QK = _QQ * _KK
_T = _H * _QK
_CHUNK = _T // _NW
_CPAD = 14416
_NVEC = _CPAD // 16
_TAB = _QQ * _H
_QBLK = 98


def _gather_indices() -> np.ndarray:
    q_blocks_h = int(np.sqrt(float(_QQ)))
    k_blocks_h = int(np.sqrt(float(_KK)))
    strides = int(np.ceil(np.sqrt(float(_KK) / float(_QQ))))
    x1, y1 = np.meshgrid(np.arange(q_blocks_h), np.arange(q_blocks_h))
    x2, y2 = np.meshgrid(np.arange(k_blocks_h), np.arange(k_blocks_h))
    aa = np.stack([x1.reshape(-1), y1.reshape(-1)], axis=-1)
    bb_grid = np.stack([x2.reshape(-1), y2.reshape(-1)], axis=-1)
    diff = np.abs(bb_grid[None, :, :] - aa[:, None, :] * strides)
    bb_pos = (diff[:, :, 0] + diff[:, :, 1] * k_blocks_h).astype(np.int64)

    f = np.arange(_T, dtype=np.int64)
    h = f // _QK
    r = f % _QK
    qc = r // (_KK * _QBLK)
    r2 = r % (_KK * _QBLK)
    k = r2 // _QBLK
    q = qc * _QBLK + r2 % _QBLK
    flat_idx = bb_pos[q, k] * _H + h
    padded = np.zeros((_NW, _CPAD), dtype=np.int32)
    padded[:, :_CHUNK] = flat_idx.reshape(_NW, _CHUNK)
    return padded


_IDX_NP = _gather_indices()


def _sc_gather_body(bb_hbm, idx_hbm, out_hbm, table_v, idx_v, vals_v):
    wid = lax.axis_index("s") * _NC + lax.axis_index("c")
    pltpu.sync_copy(bb_hbm, table_v)
    pltpu.sync_copy(idx_hbm.at[wid], idx_v)

    def body(i, carry):
        sl = pl.ds(i * 16, 16)
        vals_v[sl] = plsc.load_gather(table_v, [idx_v[sl]])
        return carry

    lax.fori_loop(0, _NVEC, body, 0)
    pltpu.sync_copy(vals_v, out_hbm.at[wid])


def _sc_gather(bb_flat, idx):
    mesh = plsc.VectorSubcoreMesh(core_axis_name="c", subcore_axis_name="s")
    fn = pl.kernel(
        _sc_gather_body,
        out_type=jax.ShapeDtypeStruct((_NW, _CPAD), jnp.float32),
        mesh=mesh,
        scratch_types=[
            pltpu.VMEM((_TAB,), jnp.float32),
            pltpu.VMEM((_CPAD,), jnp.int32),
            pltpu.VMEM((_CPAD,), jnp.float32),
        ],
        compiler_params=pltpu.CompilerParams(needs_layout_passes=False),
    )
    return fn(bb_flat, idx)


def _add_body(x_ref, b_ref, o_ref):
    for q in range(_QBLK):
        o_ref[0, q] = x_ref[0, q] + b_ref[0, 0, :, q : q + 1]


def _tc_add(x_t, bias_t, n_batch):
    return pl.pallas_call(
        _add_body,
        grid=(_H, _QQ // _QBLK),
        in_specs=[
            pl.BlockSpec((1, _QBLK, _KK, n_batch), lambda h, qc: (h, qc, 0, 0)),
            pl.BlockSpec((1, 1, _KK, _QBLK), lambda h, qc: (h, qc, 0, 0)),
        ],
        out_specs=pl.BlockSpec((1, _QBLK, _KK, n_batch), lambda h, qc: (h, qc, 0, 0)),
        out_shape=jax.ShapeDtypeStruct(x_t.shape, x_t.dtype),
    )(x_t, bias_t)


def kernel(inputs, bb):
    n_batch = inputs.shape[0]
    bb_flat = bb.reshape(-1)
    idx = jnp.asarray(_IDX_NP)
    pos = _sc_gather(bb_flat, idx)
    bias_t = pos[:, :_CHUNK].reshape(_H, _QQ // _QBLK, _KK, _QBLK)
    x_t = jnp.transpose(inputs, (1, 2, 3, 0))
    out_t = _tc_add(x_t, bias_t, n_batch)
    return jnp.transpose(out_t, (3, 0, 1, 2))


# speedup vs baseline: 6.3543x; 1.0226x over previous
import jax
import jax.numpy as jnp
import numpy as np
from jax import lax
from jax.experimental import pallas as pl
from jax.experimental.pallas import tpu as pltpu
from jax.experimental.pallas import tpu_sc as plsc

_NC = 2
_NS = 16
_NW = _NC * _NS

_QQ = 196
_KK = 196
_H = 12
_QK = _QQ * _KK
_T = _H * _QK
_CHUNK = _T // _NW
_CPAD = 14416
_NVEC = _CPAD // 16
_TAB = _QQ * _H
_QBLK = 98


def _gather_indices() -> np.ndarray:
    q_blocks_h = int(np.sqrt(float(_QQ)))
    k_blocks_h = int(np.sqrt(float(_KK)))
    strides = int(np.ceil(np.sqrt(float(_KK) / float(_QQ))))
    x1, y1 = np.meshgrid(np.arange(q_blocks_h), np.arange(q_blocks_h))
    x2, y2 = np.meshgrid(np.arange(k_blocks_h), np.arange(k_blocks_h))
    aa = np.stack([x1.reshape(-1), y1.reshape(-1)], axis=-1)
    bb_grid = np.stack([x2.reshape(-1), y2.reshape(-1)], axis=-1)
    diff = np.abs(bb_grid[None, :, :] - aa[:, None, :] * strides)
    bb_pos = (diff[:, :, 0] + diff[:, :, 1] * k_blocks_h).astype(np.int64)

    f = np.arange(_T, dtype=np.int64)
    h = f // _QK
    r = f % _QK
    qc = r // (_KK * _QBLK)
    r2 = r % (_KK * _QBLK)
    k = r2 // _QBLK
    q = qc * _QBLK + r2 % _QBLK
    flat_idx = bb_pos[q, k] * _H + h
    padded = np.zeros((_NW, _CPAD), dtype=np.int32)
    padded[:, :_CHUNK] = flat_idx.reshape(_NW, _CHUNK)
    return padded


_IDX_NP = _gather_indices()


def _sc_gather_body(bb_hbm, idx_hbm, out_hbm, table_v, idx_v, vals_v):
    wid = lax.axis_index("s") * _NC + lax.axis_index("c")
    pltpu.sync_copy(bb_hbm, table_v)
    pltpu.sync_copy(idx_hbm.at[wid], idx_v)

    @plsc.parallel_loop(0, _NVEC, unroll=8)
    def body(i):
        sl = pl.ds(i * 16, 16)
        vals_v[sl] = plsc.load_gather(table_v, [idx_v[sl]])
    pltpu.sync_copy(vals_v, out_hbm.at[wid])


def _sc_gather(bb_flat, idx):
    mesh = plsc.VectorSubcoreMesh(core_axis_name="c", subcore_axis_name="s")
    fn = pl.kernel(
        _sc_gather_body,
        out_type=jax.ShapeDtypeStruct((_NW, _CPAD), jnp.float32),
        mesh=mesh,
        scratch_types=[
            pltpu.VMEM((_TAB,), jnp.float32),
            pltpu.VMEM((_CPAD,), jnp.int32),
            pltpu.VMEM((_CPAD,), jnp.float32),
        ],
        compiler_params=pltpu.CompilerParams(needs_layout_passes=False),
    )
    return fn(bb_flat, idx)


def _add_body(x_ref, b_ref, o_ref):
    for q in range(_QBLK):
        o_ref[0, q] = x_ref[0, q] + b_ref[0, 0, :, q : q + 1]


def _tc_add(x_t, bias_t, n_batch):
    return pl.pallas_call(
        _add_body,
        grid=(_H, _QQ // _QBLK),
        in_specs=[
            pl.BlockSpec((1, _QBLK, _KK, n_batch), lambda h, qc: (h, qc, 0, 0)),
            pl.BlockSpec((1, 1, _KK, _QBLK), lambda h, qc: (h, qc, 0, 0)),
        ],
        out_specs=pl.BlockSpec((1, _QBLK, _KK, n_batch), lambda h, qc: (h, qc, 0, 0)),
        out_shape=jax.ShapeDtypeStruct(x_t.shape, x_t.dtype),
    )(x_t, bias_t)


def kernel(inputs, bb):
    n_batch = inputs.shape[0]
    bb_flat = bb.reshape(-1)
    idx = jnp.asarray(_IDX_NP)
    pos = _sc_gather(bb_flat, idx)
    bias_t = pos[:, :_CHUNK].reshape(_H, _QQ // _QBLK, _KK, _QBLK)
    x_t = jnp.transpose(inputs, (1, 2, 3, 0))
    out_t = _tc_add(x_t, bias_t, n_batch)
    return jnp.transpose(out_t, (3, 0, 1, 2))


# speedup vs baseline: 6.5033x; 1.0234x over previous
import jax
import jax.numpy as jnp
import numpy as np
from jax import lax
from jax.experimental import pallas as pl
from jax.experimental.pallas import tpu as pltpu
from jax.experimental.pallas import tpu_sc as plsc

_NC = 2
_NS = 16

_QQ = 196
_KK = 196
_H = 12
_QBLK = 98
_NQC = _QQ // _QBLK
_PLANE = _KK * _QBLK
_PPAD = 19216
_PVEC = _PPAD // 16
_TAB = _QQ * _H


def _bb_pos_planes() -> np.ndarray:
    q_blocks_h = int(np.sqrt(float(_QQ)))
    k_blocks_h = int(np.sqrt(float(_KK)))
    strides = int(np.ceil(np.sqrt(float(_KK) / float(_QQ))))
    x1, y1 = np.meshgrid(np.arange(q_blocks_h), np.arange(q_blocks_h))
    x2, y2 = np.meshgrid(np.arange(k_blocks_h), np.arange(k_blocks_h))
    aa = np.stack([x1.reshape(-1), y1.reshape(-1)], axis=-1)
    bb_grid = np.stack([x2.reshape(-1), y2.reshape(-1)], axis=-1)
    diff = np.abs(bb_grid[None, :, :] - aa[:, None, :] * strides)
    bb_pos = (diff[:, :, 0] + diff[:, :, 1] * k_blocks_h).astype(np.int64)

    p = np.arange(_PLANE, dtype=np.int64)
    k = p // _QBLK
    qp = p % _QBLK
    planes = np.zeros((_NQC, _PPAD), dtype=np.int32)
    for qc in range(_NQC):
        planes[qc, :_PLANE] = bb_pos[qc * _QBLK + qp, k]
    return planes


_IDX_NP = _bb_pos_planes()


def _sc_gather_body(bb_hbm, idx_hbm, out_hbm, table_v, idx_v, plane_v):
    wid = lax.axis_index("s") * _NC + lax.axis_index("c")

    @pl.when(wid < _H * _NQC)
    def _():
        h = wid // _NQC
        qc = wid % _NQC
        pltpu.sync_copy(bb_hbm, table_v)
        pltpu.sync_copy(idx_hbm.at[qc], idx_v)
        hoff = h * _QQ

        @plsc.parallel_loop(0, _PVEC, unroll=8)
        def body(i):
            sl = pl.ds(i * 16, 16)
            p = i * 16 + lax.iota(jnp.int32, 16)
            vals = plsc.load_gather(table_v, [idx_v[sl] + hoff])
            plsc.store_scatter(
                plane_v, [p // _QBLK, p % _QBLK], vals, mask=p < _PLANE
            )

        pltpu.sync_copy(plane_v, out_hbm.at[h, qc])


def _sc_gather(bb_t_flat, idx):
    mesh = plsc.VectorSubcoreMesh(core_axis_name="c", subcore_axis_name="s")
    fn = pl.kernel(
        _sc_gather_body,
        out_type=jax.ShapeDtypeStruct((_H, _NQC, _KK, _QBLK), jnp.float32),
        mesh=mesh,
        scratch_types=[
            pltpu.VMEM((_TAB,), jnp.float32),
            pltpu.VMEM((_PPAD,), jnp.int32),
            pltpu.VMEM((_KK, _QBLK), jnp.float32),
        ],
        compiler_params=pltpu.CompilerParams(needs_layout_passes=False),
    )
    return fn(bb_t_flat, idx)


def _add_body(x_ref, b_ref, o_ref):
    for q in range(_QBLK):
        o_ref[0, q] = x_ref[0, q] + b_ref[0, 0, :, q : q + 1]


def _tc_add(x_t, bias_t, n_batch):
    return pl.pallas_call(
        _add_body,
        grid=(_H, _NQC),
        in_specs=[
            pl.BlockSpec((1, _QBLK, _KK, n_batch), lambda h, qc: (h, qc, 0, 0)),
            pl.BlockSpec((1, 1, _KK, _QBLK), lambda h, qc: (h, qc, 0, 0)),
        ],
        out_specs=pl.BlockSpec((1, _QBLK, _KK, n_batch), lambda h, qc: (h, qc, 0, 0)),
        out_shape=jax.ShapeDtypeStruct(x_t.shape, x_t.dtype),
    )(x_t, bias_t)


def kernel(inputs, bb):
    n_batch = inputs.shape[0]
    bb_t_flat = jnp.transpose(bb, (1, 0)).reshape(-1)
    idx = jnp.asarray(_IDX_NP)
    bias_t = _sc_gather(bb_t_flat, idx)
    x_t = jnp.transpose(inputs, (1, 2, 3, 0))
    out_t = _tc_add(x_t, bias_t, n_batch)
    return jnp.transpose(out_t, (3, 0, 1, 2))


# speedup vs baseline: 6.5217x; 1.0028x over previous
import jax
import jax.numpy as jnp
import numpy as np
from jax import lax
from jax.experimental import pallas as pl
from jax.experimental.pallas import tpu as pltpu
from jax.experimental.pallas import tpu_sc as plsc

_NC = 2
_NS = 16

_QQ = 196
_KK = 196
_H = 12
_QBLK = 98
_NQC = _QQ // _QBLK
_PLANE = _KK * _QBLK
_PPAD = 19216
_PVEC = _PPAD // 16
_TAB = _QQ * _H


def _bb_pos_planes() -> np.ndarray:
    q_blocks_h = int(np.sqrt(float(_QQ)))
    k_blocks_h = int(np.sqrt(float(_KK)))
    strides = int(np.ceil(np.sqrt(float(_KK) / float(_QQ))))
    x1, y1 = np.meshgrid(np.arange(q_blocks_h), np.arange(q_blocks_h))
    x2, y2 = np.meshgrid(np.arange(k_blocks_h), np.arange(k_blocks_h))
    aa = np.stack([x1.reshape(-1), y1.reshape(-1)], axis=-1)
    bb_grid = np.stack([x2.reshape(-1), y2.reshape(-1)], axis=-1)
    diff = np.abs(bb_grid[None, :, :] - aa[:, None, :] * strides)
    bb_pos = (diff[:, :, 0] + diff[:, :, 1] * k_blocks_h).astype(np.int64)

    p = np.arange(_PLANE, dtype=np.int64)
    k = p // _QBLK
    qp = p % _QBLK
    planes = np.zeros((_NQC, _PPAD), dtype=np.int32)
    for qc in range(_NQC):
        planes[qc, :_PLANE] = bb_pos[qc * _QBLK + qp, k]
    return planes


_IDX_NP = _bb_pos_planes()


def _sc_gather_body(bb_hbm, idx_hbm, out_hbm, table_v, idx_v, plane_v):
    wid = lax.axis_index("s") * _NC + lax.axis_index("c")

    @pl.when(wid < _H * _NQC)
    def _():
        h = wid // _NQC
        qc = wid % _NQC
        pltpu.sync_copy(bb_hbm, table_v)
        pltpu.sync_copy(idx_hbm.at[qc], idx_v)
        hoff = h * _QQ

        @plsc.parallel_loop(0, _PVEC - 1, unroll=8)
        def body(i):
            sl = pl.ds(i * 16, 16)
            p = i * 16 + lax.iota(jnp.int32, 16)
            vals = plsc.load_gather(table_v, [idx_v[sl] + hoff])
            plsc.store_scatter(plane_v, [p // _QBLK, p % _QBLK], vals)

        last = _PVEC - 1
        p = last * 16 + lax.iota(jnp.int32, 16)
        vals = plsc.load_gather(table_v, [idx_v[pl.ds(last * 16, 16)] + hoff])
        plsc.store_scatter(
            plane_v, [p // _QBLK, p % _QBLK], vals, mask=p < _PLANE
        )

        pltpu.sync_copy(plane_v, out_hbm.at[h, qc])


def _sc_gather(bb_t_flat, idx):
    mesh = plsc.VectorSubcoreMesh(core_axis_name="c", subcore_axis_name="s")
    fn = pl.kernel(
        _sc_gather_body,
        out_type=jax.ShapeDtypeStruct((_H, _NQC, _KK, _QBLK), jnp.float32),
        mesh=mesh,
        scratch_types=[
            pltpu.VMEM((_TAB,), jnp.float32),
            pltpu.VMEM((_PPAD,), jnp.int32),
            pltpu.VMEM((_KK, _QBLK), jnp.float32),
        ],
        compiler_params=pltpu.CompilerParams(needs_layout_passes=False),
    )
    return fn(bb_t_flat, idx)


def _add_body(x_ref, b_ref, o_ref):
    for q in range(_QBLK):
        o_ref[0, q] = x_ref[0, q] + b_ref[0, 0, :, q : q + 1]


def _tc_add(x_t, bias_t, n_batch):
    return pl.pallas_call(
        _add_body,
        grid=(_H, _NQC),
        in_specs=[
            pl.BlockSpec((1, _QBLK, _KK, n_batch), lambda h, qc: (h, qc, 0, 0)),
            pl.BlockSpec((1, 1, _KK, _QBLK), lambda h, qc: (h, qc, 0, 0)),
        ],
        out_specs=pl.BlockSpec((1, _QBLK, _KK, n_batch), lambda h, qc: (h, qc, 0, 0)),
        out_shape=jax.ShapeDtypeStruct(x_t.shape, x_t.dtype),
    )(x_t, bias_t)


def kernel(inputs, bb):
    n_batch = inputs.shape[0]
    bb_t_flat = jnp.transpose(bb, (1, 0)).reshape(-1)
    idx = jnp.asarray(_IDX_NP)
    bias_t = _sc_gather(bb_t_flat, idx)
    x_t = jnp.transpose(inputs, (1, 2, 3, 0))
    out_t = _tc_add(x_t, bias_t, n_batch)
    return jnp.transpose(out_t, (3, 0, 1, 2))
